# Initial kernel scaffold; baseline (speedup 1.0000x reference)
import functools

import jax
import jax.numpy as jnp
from jax import lax
from jax.experimental import pallas as pl
from jax.experimental.pallas import tpu as pltpu
from jax.experimental.pallas import tpu_sc as plsc

NW = 32
CHUNK = 4
L = 16


def _sc_agg(table, nbr_flat, V, K, F2):
    rows = CHUNK * K
    nchunk = V // CHUNK
    nj = F2 // L
    mesh = plsc.VectorSubcoreMesh(core_axis_name="c", subcore_axis_name="s")

    @functools.partial(
        pl.kernel,
        out_type=jax.ShapeDtypeStruct((V, F2), jnp.float32),
        mesh=mesh,
        scratch_types=[
            pltpu.VMEM((rows,), jnp.int32),
            pltpu.VMEM((rows, F2), jnp.float32),
            pltpu.VMEM((CHUNK, F2), jnp.float32),
            pltpu.SemaphoreType.DMA,
        ],
    )
    def agg(table_hbm, nbr_hbm, out_hbm, idx_v, rows_v, outrow_v, sem):
        wid = lax.axis_index("s") * 2 + lax.axis_index("c")
        niter = (nchunk - wid + NW - 1) // NW

        def chunk_body(t, carry):
            c = wid + t * NW
            pltpu.sync_copy(nbr_hbm.at[pl.ds(c * rows, rows)], idx_v)
            pltpu.async_copy(table_hbm.at[idx_v], rows_v, sem).wait()
            for n in range(CHUNK):
                base = n * K
                accs = tuple(rows_v[base, pl.ds(j * L, L)] for j in range(nj))

                def kbody(k, a):
                    return tuple(
                        a[j] + rows_v[base + k, pl.ds(j * L, L)]
                        for j in range(nj)
                    )

                accs = lax.fori_loop(1, K, kbody, accs)
                scale = jnp.float32(1.0 / K)
                for j in range(nj):
                    outrow_v[n, pl.ds(j * L, L)] = accs[j] * scale
            pltpu.sync_copy(outrow_v, out_hbm.at[pl.ds(c * CHUNK, CHUNK)])
            return carry

        lax.fori_loop(0, niter, chunk_body, 0)

    return agg(table, nbr_flat)


def _tc_transform(x, agg, Wx, Wn, bias, blk):
    B, V, F = x.shape

    def body(x_ref, a_ref, wx_ref, wn_ref, b_ref, o_ref):
        o = jnp.dot(x_ref[0], wx_ref[...], preferred_element_type=jnp.float32)
        o += jnp.dot(a_ref[...], wn_ref[...], preferred_element_type=jnp.float32)
        o_ref[...] = (o + b_ref[...])[None]

    return pl.pallas_call(
        body,
        out_shape=jax.ShapeDtypeStruct((B, V, F), jnp.float32),
        grid=(B, V // blk),
        in_specs=[
            pl.BlockSpec((1, blk, F), lambda b, i: (b, i, 0)),
            pl.BlockSpec((blk, F), lambda b, i: (i, b)),
            pl.BlockSpec((F, F), lambda b, i: (0, 0)),
            pl.BlockSpec((F, F), lambda b, i: (0, 0)),
            pl.BlockSpec((1, F), lambda b, i: (0, 0)),
        ],
        out_specs=pl.BlockSpec((1, blk, F), lambda b, i: (b, i, 0)),
    )(x, agg, Wx, Wn, bias)


def kernel(x, neighbor, Wx, Wn, b):
    B, V, F = x.shape
    K = neighbor.shape[-1]
    table = jnp.transpose(x, (1, 0, 2)).reshape(V, B * F)
    table = jnp.concatenate([jnp.zeros((1, B * F), jnp.float32), table], axis=0)
    agg = _sc_agg(table, neighbor.reshape(-1), V, K, B * F)
    return _tc_transform(x, agg, Wx, Wn, b.reshape(1, F), 2000)


# speedup vs baseline: 6.3506x; 6.3506x over previous
import functools

import jax
import jax.numpy as jnp
from jax import lax
from jax.experimental import pallas as pl
from jax.experimental.pallas import tpu as pltpu
from jax.experimental.pallas import tpu_sc as plsc

NW = 32
CHUNK = 4
L = 16


def _sc_agg(table, nbr_flat, V, K, F2):
    rows = CHUNK * K
    nchunk = V // CHUNK
    nj = F2 // L
    mesh = plsc.VectorSubcoreMesh(core_axis_name="c", subcore_axis_name="s")

    @functools.partial(
        pl.kernel,
        out_type=jax.ShapeDtypeStruct((V, F2), jnp.float32),
        mesh=mesh,
        scratch_types=[
            pltpu.VMEM((rows,), jnp.int32),
            pltpu.VMEM((rows,), jnp.int32),
            pltpu.VMEM((rows, F2), jnp.float32),
            pltpu.VMEM((rows, F2), jnp.float32),
            pltpu.VMEM((CHUNK, F2), jnp.float32),
            pltpu.SemaphoreType.DMA,
            pltpu.SemaphoreType.DMA,
        ],
    )
    def agg(table_hbm, nbr_hbm, out_hbm, idx0, idx1, rows0, rows1,
            outrow_v, sem0, sem1):
        wid = lax.axis_index("s") * 2 + lax.axis_index("c")
        niter = (nchunk - wid + NW - 1) // NW
        bufs = ((idx0, rows0, sem0), (idx1, rows1, sem1))

        def chunk_of(t):
            return wid + t * NW

        def start(buf, t):
            idx_v, rows_v, sem = buf
            c = chunk_of(t)
            pltpu.sync_copy(nbr_hbm.at[pl.ds(c * rows, rows)], idx_v)
            pltpu.async_copy(table_hbm.at[idx_v], rows_v, sem)

        def finish(buf, t):
            idx_v, rows_v, sem = buf
            c = chunk_of(t)
            pltpu.make_async_copy(table_hbm.at[idx_v], rows_v, sem).wait()
            for n in range(CHUNK):
                base = n * K
                accs = tuple(rows_v[base, pl.ds(j * L, L)] for j in range(nj))

                def kbody(k, a):
                    return tuple(
                        a[j] + rows_v[base + k, pl.ds(j * L, L)]
                        for j in range(nj)
                    )

                accs = lax.fori_loop(1, K, kbody, accs)
                scale = jnp.float32(1.0 / K)
                for j in range(nj):
                    outrow_v[n, pl.ds(j * L, L)] = accs[j] * scale
            pltpu.sync_copy(outrow_v, out_hbm.at[pl.ds(c * CHUNK, CHUNK)])

        start(bufs[0], 0)

        def pair_body(p, carry):
            t = p * 2
            start(bufs[1], t + 1)
            finish(bufs[0], t)

            @pl.when(t + 2 < niter)
            def _():
                start(bufs[0], t + 2)

            finish(bufs[1], t + 1)
            return carry

        lax.fori_loop(0, niter // 2, pair_body, 0)

        @pl.when(niter % 2 == 1)
        def _():
            finish(bufs[0], niter - 1)

    return agg(table, nbr_flat)


def _tc_transform(x, agg, Wx, Wn, bias, blk):
    B, V, F = x.shape

    def body(x_ref, a_ref, wx_ref, wn_ref, b_ref, o_ref):
        o = jnp.dot(x_ref[0], wx_ref[...], preferred_element_type=jnp.float32)
        o += jnp.dot(a_ref[...], wn_ref[...], preferred_element_type=jnp.float32)
        o_ref[...] = (o + b_ref[...])[None]

    return pl.pallas_call(
        body,
        out_shape=jax.ShapeDtypeStruct((B, V, F), jnp.float32),
        grid=(B, V // blk),
        in_specs=[
            pl.BlockSpec((1, blk, F), lambda b, i: (b, i, 0)),
            pl.BlockSpec((blk, F), lambda b, i: (i, b)),
            pl.BlockSpec((F, F), lambda b, i: (0, 0)),
            pl.BlockSpec((F, F), lambda b, i: (0, 0)),
            pl.BlockSpec((1, F), lambda b, i: (0, 0)),
        ],
        out_specs=pl.BlockSpec((1, blk, F), lambda b, i: (b, i, 0)),
    )(x, agg, Wx, Wn, bias)


def kernel(x, neighbor, Wx, Wn, b):
    B, V, F = x.shape
    K = neighbor.shape[-1]
    table = jnp.transpose(x, (1, 0, 2)).reshape(V, B * F)
    table = jnp.concatenate([jnp.zeros((1, B * F), jnp.float32), table], axis=0)
    agg = _sc_agg(table, neighbor.reshape(-1), V, K, B * F)
    return _tc_transform(x, agg, Wx, Wn, b.reshape(1, F), 2000)
